# baseline (device time: 25363 ns/iter reference)
import jax
import jax.numpy as jnp
from jax import lax
from jax.experimental import pallas as pl
from jax.experimental.pallas import tpu as pltpu

N_GLOBAL = 2048
EPS = 1e-5
C = 512
NQ = 4


def kernel(x, gamma, beta):
    m, n_loc = x.shape
    nc = m // C
    qc = nc // NQ

    def body(x_hbm, gb_hbm, out_ref, x_vmem, gb_vmem,
             send_stats, recv_stats, in_sems, gb_sem, send_sems, recv_sems):
        my_x = lax.axis_index("x")
        my_y = lax.axis_index("y")
        peer = (my_x, 1 - my_y)

        copies_in = []
        for i in range(nc):
            cp = pltpu.make_async_copy(
                x_hbm.at[pl.ds(i * C, C), :],
                x_vmem.at[pl.ds(i * C, C), :],
                in_sems.at[i],
            )
            cp.start()
            copies_in.append(cp)
        gb_cp = pltpu.make_async_copy(gb_hbm, gb_vmem, gb_sem)
        gb_cp.start()

        barrier_sem = pltpu.get_barrier_semaphore()
        pl.semaphore_signal(
            barrier_sem, inc=1, device_id=peer,
            device_id_type=pl.DeviceIdType.MESH,
        )

        rdmas = []
        for i in range(nc):
            q, j = divmod(i, qc)
            with jax.named_scope(f"in_wait#c={i}"):
                copies_in[i].wait()
            with jax.named_scope(f"stats#c={i}"):
                xc = x_vmem[pl.ds(i * C, C), :]
                send_stats[q, :, 2 * j:2 * j + 1] = jnp.sum(
                    xc, axis=1, keepdims=True)
                send_stats[q, :, 2 * j + 1:2 * j + 2] = jnp.sum(
                    xc * xc, axis=1, keepdims=True)
            if j == qc - 1:
                if q == 0:
                    with jax.named_scope("barrier_wait"):
                        pl.semaphore_wait(barrier_sem, 1)
                rdma = pltpu.make_async_remote_copy(
                    src_ref=send_stats.at[q],
                    dst_ref=recv_stats.at[q],
                    send_sem=send_sems.at[q],
                    recv_sem=recv_sems.at[q],
                    device_id=peer,
                    device_id_type=pl.DeviceIdType.MESH,
                )
                rdma.start()
                rdmas.append(rdma)

        with jax.named_scope("gb_wait"):
            gb_cp.wait()

        for i in range(nc):
            q, j = divmod(i, qc)
            ds = pl.ds(i * C, C)
            if j == 0:
                with jax.named_scope(f"recv_wait#q={q}"):
                    rdmas[q].wait_recv()
            with jax.named_scope(f"norm#c={i}"):
                tot1 = (send_stats[q, :, 2 * j:2 * j + 1]
                        + recv_stats[q, :, 2 * j:2 * j + 1])
                tot2 = (send_stats[q, :, 2 * j + 1:2 * j + 2]
                        + recv_stats[q, :, 2 * j + 1:2 * j + 2])
                mean_c = tot1 / N_GLOBAL
                var_c = tot2 / N_GLOBAL - mean_c * mean_c
                rstd_c = lax.rsqrt(var_c + EPS)
                shift_c = -mean_c * rstd_c
                t = x_vmem[ds, :] * rstd_c + shift_c
                out_ref[ds, :] = t * gb_vmem[0:1, :] + gb_vmem[1:2, :]

        with jax.named_scope("drain"):
            for q in range(NQ):
                rdmas[q].wait_send()

    gb = jnp.stack([gamma, beta], axis=0)
    return pl.pallas_call(
        body,
        out_shape=jax.ShapeDtypeStruct((m, n_loc), jnp.float32),
        in_specs=[
            pl.BlockSpec(memory_space=pl.ANY),
            pl.BlockSpec(memory_space=pl.ANY),
        ],
        out_specs=pl.BlockSpec(memory_space=pltpu.VMEM),
        scratch_shapes=[
            pltpu.VMEM((m, n_loc), jnp.float32),
            pltpu.VMEM((2, n_loc), jnp.float32),
            pltpu.VMEM((NQ, C, 2 * qc), jnp.float32),
            pltpu.VMEM((NQ, C, 2 * qc), jnp.float32),
            pltpu.SemaphoreType.DMA((nc,)),
            pltpu.SemaphoreType.DMA,
            pltpu.SemaphoreType.DMA((NQ,)),
            pltpu.SemaphoreType.DMA((NQ,)),
        ],
        compiler_params=pltpu.CompilerParams(collective_id=0),
    )(x, gb)


# device time: 22092 ns/iter; 1.1481x vs baseline; 1.1481x over previous
import jax
import jax.numpy as jnp
from jax import lax
from jax.experimental import pallas as pl
from jax.experimental.pallas import tpu as pltpu

N_GLOBAL = 2048
EPS = 1e-5
C = 512
NQ = 2


def kernel(x, gamma, beta):
    m, n_loc = x.shape
    nc = m // C
    qc = nc // NQ

    def body(x_hbm, gb_hbm, out_ref, x_vmem, gb_vmem,
             send_stats, recv_stats, in_sems, gb_sem, send_sems, recv_sems):
        my_x = lax.axis_index("x")
        my_y = lax.axis_index("y")
        peer = (my_x, 1 - my_y)

        copies_in = []
        for i in range(nc):
            cp = pltpu.make_async_copy(
                x_hbm.at[pl.ds(i * C, C), :],
                x_vmem.at[pl.ds(i * C, C), :],
                in_sems.at[i],
            )
            cp.start()
            copies_in.append(cp)
        gb_cp = pltpu.make_async_copy(gb_hbm, gb_vmem, gb_sem)
        gb_cp.start()

        barrier_sem = pltpu.get_barrier_semaphore()
        pl.semaphore_signal(
            barrier_sem, inc=1, device_id=peer,
            device_id_type=pl.DeviceIdType.MESH,
        )

        rdmas = []
        for i in range(nc):
            q, j = divmod(i, qc)
            with jax.named_scope(f"in_wait#c={i}"):
                copies_in[i].wait()
            with jax.named_scope(f"stats#c={i}"):
                xc = x_vmem[pl.ds(i * C, C), :]
                send_stats[q, :, 2 * j:2 * j + 1] = jnp.sum(
                    xc, axis=1, keepdims=True)
                send_stats[q, :, 2 * j + 1:2 * j + 2] = jnp.sum(
                    xc * xc, axis=1, keepdims=True)
            if j == qc - 1:
                if q == 0:
                    with jax.named_scope("barrier_wait"):
                        pl.semaphore_wait(barrier_sem, 1)
                rdma = pltpu.make_async_remote_copy(
                    src_ref=send_stats.at[q],
                    dst_ref=recv_stats.at[q],
                    send_sem=send_sems.at[q],
                    recv_sem=recv_sems.at[q],
                    device_id=peer,
                    device_id_type=pl.DeviceIdType.MESH,
                )
                rdma.start()
                rdmas.append(rdma)

        with jax.named_scope("gb_wait"):
            gb_cp.wait()

        for i in range(nc):
            q, j = divmod(i, qc)
            ds = pl.ds(i * C, C)
            if j == 0:
                with jax.named_scope(f"recv_wait#q={q}"):
                    rdmas[q].wait_recv()
            with jax.named_scope(f"norm#c={i}"):
                tot1 = (send_stats[q, :, 2 * j:2 * j + 1]
                        + recv_stats[q, :, 2 * j:2 * j + 1])
                tot2 = (send_stats[q, :, 2 * j + 1:2 * j + 2]
                        + recv_stats[q, :, 2 * j + 1:2 * j + 2])
                mean_c = tot1 / N_GLOBAL
                var_c = tot2 / N_GLOBAL - mean_c * mean_c
                rstd_c = lax.rsqrt(var_c + EPS)
                shift_c = -mean_c * rstd_c
                t = x_vmem[ds, :] * rstd_c + shift_c
                out_ref[ds, :] = t * gb_vmem[0:1, :] + gb_vmem[1:2, :]

        with jax.named_scope("drain"):
            for q in range(NQ):
                rdmas[q].wait_send()

    gb = jnp.stack([gamma, beta], axis=0)
    return pl.pallas_call(
        body,
        out_shape=jax.ShapeDtypeStruct((m, n_loc), jnp.float32),
        in_specs=[
            pl.BlockSpec(memory_space=pl.ANY),
            pl.BlockSpec(memory_space=pl.ANY),
        ],
        out_specs=pl.BlockSpec(memory_space=pltpu.VMEM),
        scratch_shapes=[
            pltpu.VMEM((m, n_loc), jnp.float32),
            pltpu.VMEM((2, n_loc), jnp.float32),
            pltpu.VMEM((NQ, C, 2 * qc), jnp.float32),
            pltpu.VMEM((NQ, C, 2 * qc), jnp.float32),
            pltpu.SemaphoreType.DMA((nc,)),
            pltpu.SemaphoreType.DMA,
            pltpu.SemaphoreType.DMA((NQ,)),
            pltpu.SemaphoreType.DMA((NQ,)),
        ],
        compiler_params=pltpu.CompilerParams(collective_id=0),
    )(x, gb)
